# baseline (speedup 1.0000x reference)
import functools

import jax
import jax.numpy as jnp
from jax import lax
from jax.experimental import pallas as pl
from jax.experimental.pallas import tpu as pltpu
from jax.experimental.pallas import tpu_sc as plsc

_ROWS = 8192
_COLS = 1024
_NC = 2
_NS = 16
_NW = _NC * _NS
_LANES = 16

_SC_ROWS = 2560
_TC_ROWS = _ROWS - _SC_ROWS
_ROWS_W = _SC_ROWS // _NW
_CR = 16
_NCHUNKS = _ROWS_W // _CR
_GROUPS = _CR * _COLS // _LANES
_TC_BLOCK = 512

_mesh = plsc.VectorSubcoreMesh(core_axis_name="c", subcore_axis_name="s")


@functools.partial(
    pl.kernel,
    out_type=jax.ShapeDtypeStruct((_SC_ROWS, _COLS), jnp.float32),
    mesh=_mesh,
    scratch_types=[
        pltpu.VMEM((2, _CR, _COLS), jnp.float32),
        pltpu.VMEM((2, _CR, _COLS), jnp.float32),
        pltpu.VMEM((2, _CR, _COLS), jnp.float32),
        pltpu.SemaphoreType.DMA((2,)),
        pltpu.SemaphoreType.DMA((2,)),
    ],
)
def _sc_add(x_hbm, p_hbm, out_hbm, xbuf, pbuf, obuf, sin, sout):
    wid = lax.axis_index("s") * _NC + lax.axis_index("c")
    base = wid * _ROWS_W

    def start_in(k, b):
        off = base + k * _CR
        pltpu.async_copy(x_hbm.at[pl.ds(_TC_ROWS + off, _CR)], xbuf.at[b], sin.at[b])
        pltpu.async_copy(p_hbm.at[pl.ds(_TC_ROWS + off, _CR)], pbuf.at[b], sin.at[b])

    def wait_in(k, b):
        off = base + k * _CR
        pltpu.make_async_copy(x_hbm.at[pl.ds(_TC_ROWS + off, _CR)], xbuf.at[b], sin.at[b]).wait()
        pltpu.make_async_copy(p_hbm.at[pl.ds(_TC_ROWS + off, _CR)], pbuf.at[b], sin.at[b]).wait()

    def start_out(k, b):
        off = base + k * _CR
        pltpu.async_copy(obuf.at[b], out_hbm.at[pl.ds(off, _CR)], sout.at[b])

    def wait_out(k, b):
        off = base + k * _CR
        pltpu.make_async_copy(obuf.at[b], out_hbm.at[pl.ds(off, _CR)], sout.at[b]).wait()

    start_in(0, 0)

    def chunk_body(k, carry):
        b = lax.rem(k, 2)

        def slot_body(b):
            wait_in(k, b)

            @pl.when(k + 1 < _NCHUNKS)
            def _():
                start_in(k + 1, 1 - b)

            @pl.when(k >= 2)
            def _():
                wait_out(k - 2, b)

            def add_group(i):
                r = lax.shift_right_logical(i, 6)
                c = lax.shift_left(lax.bitwise_and(i, 63), 4)
                s = pl.ds(pl.multiple_of(c, _LANES), _LANES)
                obuf[b, r, s] = xbuf[b, r, s] + pbuf[b, r, s]

            plsc.parallel_loop(0, _GROUPS, 1, unroll=8)(add_group)
            start_out(k, b)

        @pl.when(b == 0)
        def _():
            slot_body(0)

        @pl.when(b == 1)
        def _():
            slot_body(1)

        return carry

    lax.fori_loop(0, _NCHUNKS, chunk_body, 0)
    wait_out(_NCHUNKS - 2, (_NCHUNKS - 2) % 2)
    wait_out(_NCHUNKS - 1, (_NCHUNKS - 1) % 2)


def _tc_block(x_ref, p_ref, o_ref):
    o_ref[...] = x_ref[...] + p_ref[...]


def _tc_add(x, pos_table):
    spec = pl.BlockSpec((_TC_BLOCK, _COLS), lambda i: (i, 0))
    return pl.pallas_call(
        _tc_block,
        grid=(_TC_ROWS // _TC_BLOCK,),
        in_specs=[spec, spec],
        out_specs=spec,
        out_shape=jax.ShapeDtypeStruct((_TC_ROWS, _COLS), jnp.float32),
    )(x, pos_table)


def kernel(x, pos_table):
    n = x.shape[0]
    p = pos_table[:n]
    top = _tc_add(x, p)
    bot = _sc_add(x, p)
    return jnp.concatenate([top, bot], axis=0)


# speedup vs baseline: 1.2230x; 1.2230x over previous
import functools

import jax
import jax.numpy as jnp
from jax import lax
from jax.experimental import pallas as pl
from jax.experimental.pallas import tpu as pltpu
from jax.experimental.pallas import tpu_sc as plsc

_ROWS = 8192
_COLS = 1024
_NC = 2
_NS = 16
_NW = _NC * _NS
_ROWS_W = _ROWS // _NW
_CR = 16
_NCHUNKS = _ROWS_W // _CR
_LANES = 16
_GROUPS = _CR * _COLS // _LANES

_mesh = plsc.VectorSubcoreMesh(core_axis_name="c", subcore_axis_name="s")


@functools.partial(
    pl.kernel,
    out_type=jax.ShapeDtypeStruct((_ROWS, _COLS), jnp.float32),
    mesh=_mesh,
    scratch_types=[
        pltpu.VMEM((2, _CR, _COLS), jnp.float32),
        pltpu.VMEM((2, _CR, _COLS), jnp.float32),
        pltpu.VMEM((2, _CR, _COLS), jnp.float32),
        pltpu.SemaphoreType.DMA((2,)),
        pltpu.SemaphoreType.DMA((2,)),
    ],
)
def _sc_add(x_hbm, p_hbm, out_hbm, xbuf, pbuf, obuf, sin, sout):
    wid = lax.axis_index("s") * _NC + lax.axis_index("c")

    def start_in(k, b):
        off = (k * _NW + wid) * _CR
        pltpu.async_copy(x_hbm.at[pl.ds(off, _CR)], xbuf.at[b], sin.at[b])
        pltpu.async_copy(p_hbm.at[pl.ds(off, _CR)], pbuf.at[b], sin.at[b])

    def wait_in(k, b):
        off = (k * _NW + wid) * _CR
        pltpu.make_async_copy(x_hbm.at[pl.ds(off, _CR)], xbuf.at[b], sin.at[b]).wait()
        pltpu.make_async_copy(p_hbm.at[pl.ds(off, _CR)], pbuf.at[b], sin.at[b]).wait()

    def start_out(k, b):
        off = (k * _NW + wid) * _CR
        pltpu.async_copy(obuf.at[b], out_hbm.at[pl.ds(off, _CR)], sout.at[b])

    def wait_out(k, b):
        off = (k * _NW + wid) * _CR
        pltpu.make_async_copy(obuf.at[b], out_hbm.at[pl.ds(off, _CR)], sout.at[b]).wait()

    start_in(0, 0)

    def pair_body(k2, carry):
        for b in range(2):
            k = 2 * k2 + b
            wait_in(k, b)

            @pl.when(k + 1 < _NCHUNKS)
            def _():
                start_in(k + 1, 1 - b)

            @pl.when(k >= 2)
            def _():
                wait_out(k - 2, b)

            def add_group(i):
                r = lax.shift_right_logical(i, 6)
                c = lax.shift_left(lax.bitwise_and(i, 63), 4)
                s = pl.ds(pl.multiple_of(c, _LANES), _LANES)
                obuf[b, r, s] = xbuf[b, r, s] + pbuf[b, r, s]

            plsc.parallel_loop(0, _GROUPS, 1, unroll=8)(add_group)
            start_out(k, b)
        return carry

    lax.fori_loop(0, _NCHUNKS // 2, pair_body, 0)
    wait_out(_NCHUNKS - 2, 0)
    wait_out(_NCHUNKS - 1, 1)


def kernel(x, pos_table):
    n = x.shape[0]
    return _sc_add(x, pos_table[:n])


# speedup vs baseline: 1.3130x; 1.0736x over previous
import functools

import jax
import jax.numpy as jnp
from jax import lax
from jax.experimental import pallas as pl
from jax.experimental.pallas import tpu as pltpu
from jax.experimental.pallas import tpu_sc as plsc

_ROWS = 8192
_COLS = 1024
_NC = 2
_NS = 16
_NW = _NC * _NS
_ROWS_W = _ROWS // _NW
_CR = 8
_NB = 4
_NCHUNKS = _ROWS_W // _CR
_LANES = 16
_GROUPS = _CR * _COLS // _LANES

_mesh = plsc.VectorSubcoreMesh(core_axis_name="c", subcore_axis_name="s")


@functools.partial(
    pl.kernel,
    out_type=jax.ShapeDtypeStruct((_ROWS, _COLS), jnp.float32),
    mesh=_mesh,
    scratch_types=[
        pltpu.VMEM((_NB, _CR, _COLS), jnp.float32),
        pltpu.VMEM((_NB, _CR, _COLS), jnp.float32),
        pltpu.VMEM((_NB, _CR, _COLS), jnp.float32),
        pltpu.SemaphoreType.DMA((_NB,)),
        pltpu.SemaphoreType.DMA((_NB,)),
    ],
)
def _sc_add(x_hbm, p_hbm, out_hbm, xbuf, pbuf, obuf, sin, sout):
    wid = lax.axis_index("s") * _NC + lax.axis_index("c")
    base = wid * _ROWS_W

    def start_in(k, b):
        off = base + k * _CR
        pltpu.async_copy(x_hbm.at[pl.ds(off, _CR)], xbuf.at[b], sin.at[b])
        pltpu.async_copy(p_hbm.at[pl.ds(off, _CR)], pbuf.at[b], sin.at[b])

    def wait_in(k, b):
        off = base + k * _CR
        pltpu.make_async_copy(x_hbm.at[pl.ds(off, _CR)], xbuf.at[b], sin.at[b]).wait()
        pltpu.make_async_copy(p_hbm.at[pl.ds(off, _CR)], pbuf.at[b], sin.at[b]).wait()

    def start_out(k, b):
        off = base + k * _CR
        pltpu.async_copy(obuf.at[b], out_hbm.at[pl.ds(off, _CR)], sout.at[b])

    def wait_out(k, b):
        off = base + k * _CR
        pltpu.make_async_copy(obuf.at[b], out_hbm.at[pl.ds(off, _CR)], sout.at[b]).wait()

    for b in range(_NB - 1):
        start_in(b, b)

    def quad_body(k4, carry):
        for b in range(_NB):
            k = _NB * k4 + b
            wait_in(k, b)

            @pl.when(k + _NB - 1 < _NCHUNKS)
            def _():
                start_in(k + _NB - 1, (b + _NB - 1) % _NB)

            @pl.when(k >= _NB)
            def _():
                wait_out(k - _NB, b)

            def add_group(i):
                r = lax.shift_right_logical(i, 6)
                c = lax.shift_left(lax.bitwise_and(i, 63), 4)
                s = pl.ds(pl.multiple_of(c, _LANES), _LANES)
                obuf[b, r, s] = xbuf[b, r, s] + pbuf[b, r, s]

            plsc.parallel_loop(0, _GROUPS, 1, unroll=8)(add_group)
            start_out(k, b)
        return carry

    lax.fori_loop(0, _NCHUNKS // _NB, quad_body, 0)
    for k in range(_NCHUNKS - _NB, _NCHUNKS):
        wait_out(k, k % _NB)


def kernel(x, pos_table):
    n = x.shape[0]
    return _sc_add(x, pos_table[:n])
